# baseline (speedup 1.0000x reference)
import functools

import jax
import jax.numpy as jnp
from jax import lax
from jax.experimental import pallas as pl
from jax.experimental.pallas import tpu as pltpu
from jax.experimental.pallas import tpu_sc as plsc

B = 16384
D = 64
K = 5
NC = 2
NS = 16
NW = NC * NS
BPW = B // NW
C = 64
NCH = BPW // C


def _worker_id():
    return lax.axis_index("s") * NC + lax.axis_index("c")


def _sc_body(pos_u_hbm, pos_v_hbm, neg_t_hbm, in2_hbm, out2_hbm,
             ps_hbm, ns_hbm,
             idx_u, idx_v, idx_n, hx_u, hx_vn,
             rows_u, rows_vn, ps_buf, ns_buf,
             isem, sem0, sem1):
    wid = _worker_id()
    base = pl.multiple_of(wid * BPW, BPW)
    lane = lax.iota(jnp.int32, 16)
    sems = (sem0, sem1)

    icopies = [
        pltpu.async_copy(pos_u_hbm.at[pl.ds(base, BPW)], idx_u, isem),
        pltpu.async_copy(pos_v_hbm.at[pl.ds(base, BPW)], idx_v, isem),
    ]
    for k in range(K):
        icopies.append(
            pltpu.async_copy(neg_t_hbm.at[pl.ds(k * B + base, BPW)],
                             idx_n.at[pl.ds(k * BPW, BPW)], isem))
    for cp in icopies:
        cp.wait()

    def hmap(t):
        return ((t >> RSH) << (RSH - 2)) | (t & (RB // 4 - 1))

    for i in range(BPW // 16):
        sl = pl.ds(16 * i, 16)
        hx_u[sl] = hmap(idx_u[sl])
    for c in range(NCH):
        for i in range(C // 16):
            hx_vn[pl.ds(c * 6 * C + 16 * i, 16)] = (
                hmap(idx_v[pl.ds(c * C + 16 * i, 16)]))
            for k in range(K):
                hx_vn[pl.ds(c * 6 * C + (k + 1) * C + 16 * i, 16)] = (
                    hmap(idx_n[pl.ds(k * BPW + c * C + 16 * i, 16)]))

    def fire(c, s):
        cb = pl.multiple_of((c % NCH) * C, C)
        pltpu.async_copy(in2_hbm.at[hx_u.at[pl.ds(cb, C)]], rows_u[s],
                         sems[s])
        pltpu.async_copy(out2_hbm.at[hx_vn.at[pl.ds(6 * cb, 6 * C)]],
                         rows_vn[s], sems[s])

    def drain(s):
        pltpu.make_async_copy(in2_hbm.at[hx_u.at[pl.ds(0, C)]], rows_u[s],
                              sems[s]).wait()
        pltpu.make_async_copy(out2_hbm.at[hx_vn.at[pl.ds(0, 6 * C)]],
                              rows_vn[s], sems[s]).wait()

    fire(0, 0)

    def chunk_pair(cc, carry):
        for bset in range(2):
            c = cc * 2 + bset
            fire(c + 1, 1 - bset)
            drain(bset)

            def group_body(g, gcarry):
                row0 = g * 16
                goff = pl.multiple_of(c * C + row0, 16)
                sl = pl.ds(goff, 16)
                pu = ((idx_u[sl] >> (RSH - 2)) & 3) * 32
                pv = ((idx_v[sl] >> (RSH - 2)) & 3) * 32
                pn = [((idx_n[pl.ds(k * BPW + goff, 16)] >> (RSH - 2)) & 3)
                      * 32 for k in range(K)]
                accp = jnp.zeros((16,), jnp.float32)
                accn = [jnp.zeros((16,), jnp.float32) for _ in range(K)]
                def load4(ref, row, bcol):
                    out = []
                    for j in range(2):
                        raw = ref[row, pl.ds(bcol + 16 * j, 16)]
                        bf = plsc.bitcast(raw, jnp.bfloat16)
                        out.extend(plsc.unpack(
                            bf, format=plsc.PackFormat.INTERLEAVED))
                    return out

                for rr in range(16):
                    r = row0 + rr
                    u = load4(rows_u[bset], r, pu[rr])
                    v = load4(rows_vn[bset], r, pv[rr])
                    s = u[0] * v[0] + u[1] * v[1] + u[2] * v[2] + u[3] * v[3]
                    accp = jnp.where(lane == rr, jnp.sum(s), accp)
                    for k in range(K):
                        n = load4(rows_vn[bset], (k + 1) * C + r, pn[k][rr])
                        t = (u[0] * n[0] + u[1] * n[1] + u[2] * n[2]
                             + u[3] * n[3])
                        accn[k] = jnp.where(lane == rr, jnp.sum(t), accn[k])
                ps_buf[sl] = accp
                for k in range(K):
                    ns_buf[pl.ds(k * BPW + goff, 16)] = accn[k]
                return gcarry

            lax.fori_loop(0, C // 16, group_body, 0, unroll=False)
        return carry

    lax.fori_loop(0, NCH // 2, chunk_pair, 0, unroll=False)
    drain(0)

    pltpu.sync_copy(ps_buf, ps_hbm.at[pl.ds(base, BPW)])
    for k in range(K):
        pltpu.sync_copy(ns_buf.at[pl.ds(k * BPW, BPW)],
                        ns_hbm.at[pl.ds(k * B + base, BPW)])


@functools.partial(
    pl.kernel,
    out_type=[
        jax.ShapeDtypeStruct((B,), jnp.float32),
        jax.ShapeDtypeStruct((K * B,), jnp.float32),
    ],
    mesh=plsc.VectorSubcoreMesh(core_axis_name="c", subcore_axis_name="s",
                                num_cores=NC, num_subcores=NS),
    compiler_params=pltpu.CompilerParams(needs_layout_passes=False),
    scratch_types=[
        pltpu.VMEM((BPW,), jnp.int32),
        pltpu.VMEM((BPW,), jnp.int32),
        pltpu.VMEM((K * BPW,), jnp.int32),
        pltpu.VMEM((BPW,), jnp.int32),
        pltpu.VMEM((6 * BPW,), jnp.int32),
        [pltpu.VMEM((C, 2 * D), jnp.int32) for _ in range(2)],
        [pltpu.VMEM((6 * C, 2 * D), jnp.int32) for _ in range(2)],
        pltpu.VMEM((BPW,), jnp.float32),
        pltpu.VMEM((K * BPW,), jnp.float32),
        pltpu.SemaphoreType.DMA,
        pltpu.SemaphoreType.DMA,
        pltpu.SemaphoreType.DMA,
    ],
)
def _sc_scores(pos_u_hbm, pos_v_hbm, neg_t_hbm, in2_hbm, out2_hbm,
               ps_hbm, ns_hbm, *scratch):
    _sc_body(pos_u_hbm, pos_v_hbm, neg_t_hbm, in2_hbm, out2_hbm,
             ps_hbm, ns_hbm, *scratch)


RB = 65536
RH = RB // 2
RSH = RB.bit_length() - 1


RQ = RB // 4


def _relayout_body(xt_ref, o_ref):
    x = xt_ref[...]
    xr = lax.bitcast_convert_type(x, jnp.uint32).reshape(D // 2, 2, RB)
    half = jnp.uint32(0x8000)
    lo = (xr[:, 0, :] + half) >> 16
    hi = (xr[:, 1, :] + half) & jnp.uint32(0xFFFF0000)
    w = lax.bitcast_convert_type(lo | hi, jnp.int32)
    o_ref[...] = jnp.concatenate(
        [w[:, q * RQ:(q + 1) * RQ] for q in range(4)], axis=0).T


def _tc_relayout(xt):
    v = xt.shape[1]
    nb = pl.cdiv(v, RB)
    return pl.pallas_call(
        _relayout_body,
        grid=(nb,),
        in_specs=[pl.BlockSpec((D, RB), lambda i: (0, i))],
        out_specs=pl.BlockSpec((RQ, 2 * D), lambda i: (i, 0)),
        out_shape=jax.ShapeDtypeStruct((nb * RQ, 2 * D), jnp.int32),
        compiler_params=pltpu.CompilerParams(
            vmem_limit_bytes=100 * 1024 * 1024),
    )(xt)


def _loss_body(ps_ref, ns_ref, out_ref):
    p = ps_ref[...]
    n = ns_ref[...]
    lsp = jax.nn.log_sigmoid(p)
    lsn = jax.nn.log_sigmoid(-n)
    total = jnp.sum(lsp) + jnp.sum(lsn)
    out_ref[0, 0] = -(total / B)


def _tc_loss(ps2d, ns2d):
    return pl.pallas_call(
        _loss_body,
        out_shape=jax.ShapeDtypeStruct((1, 1), jnp.float32),
        out_specs=pl.BlockSpec(memory_space=pltpu.SMEM),
    )(ps2d, ns2d)


def kernel(pos_u, pos_v, neg_v, in_embed, out_embed):
    neg_t = neg_v.astype(jnp.int32).T.reshape(K * B)
    in2 = _tc_relayout(in_embed.T)
    out2 = _tc_relayout(out_embed.T)
    ps, ns = _sc_scores(pos_u.astype(jnp.int32), pos_v.astype(jnp.int32),
                        neg_t, in2, out2)
    loss = _tc_loss(ps.reshape(B // 128, 128), ns.reshape(K * B // 128, 128))
    return loss[0, 0]


# speedup vs baseline: 1.0423x; 1.0423x over previous
import functools

import jax
import jax.numpy as jnp
from jax import lax
from jax.experimental import pallas as pl
from jax.experimental.pallas import tpu as pltpu
from jax.experimental.pallas import tpu_sc as plsc

B = 16384
D = 64
K = 5
NC = 2
NS = 16
NW = NC * NS
BPW = B // NW
C = 64
NCH = BPW // C


def _worker_id():
    return lax.axis_index("s") * NC + lax.axis_index("c")


def _sc_body(pos_u_hbm, pos_v_hbm, neg_t_hbm, in2_hbm, out2_hbm,
             ps_hbm, ns_hbm,
             idx_u, idx_v, idx_n, hx_u, hx_vn,
             rows_u, rows_vn, ps_buf, ns_buf,
             isem, sem0, sem1):
    wid = _worker_id()
    base = pl.multiple_of(wid * BPW, BPW)
    lane = lax.iota(jnp.int32, 16)
    sems = (sem0, sem1)

    icopies = [
        pltpu.async_copy(pos_u_hbm.at[pl.ds(base, BPW)], idx_u, isem),
        pltpu.async_copy(pos_v_hbm.at[pl.ds(base, BPW)], idx_v, isem),
    ]
    for k in range(K):
        icopies.append(
            pltpu.async_copy(neg_t_hbm.at[pl.ds(k * B + base, BPW)],
                             idx_n.at[pl.ds(k * BPW, BPW)], isem))
    for cp in icopies:
        cp.wait()

    def hmap(t):
        return ((t >> RSH) << (RSH - 1)) | (t & (RH - 1))

    for i in range(BPW // 16):
        sl = pl.ds(16 * i, 16)
        hx_u[sl] = hmap(idx_u[sl])
    for c in range(NCH):
        for i in range(C // 16):
            hx_vn[pl.ds(c * 6 * C + 16 * i, 16)] = (
                hmap(idx_v[pl.ds(c * C + 16 * i, 16)]))
            for k in range(K):
                hx_vn[pl.ds(c * 6 * C + (k + 1) * C + 16 * i, 16)] = (
                    hmap(idx_n[pl.ds(k * BPW + c * C + 16 * i, 16)]))

    def fire(c, s):
        cb = pl.multiple_of((c % NCH) * C, C)
        pltpu.async_copy(in2_hbm.at[hx_u.at[pl.ds(cb, C)]], rows_u[s],
                         sems[s])
        pltpu.async_copy(out2_hbm.at[hx_vn.at[pl.ds(6 * cb, 6 * C)]],
                         rows_vn[s], sems[s])

    def drain(s):
        pltpu.make_async_copy(in2_hbm.at[hx_u.at[pl.ds(0, C)]], rows_u[s],
                              sems[s]).wait()
        pltpu.make_async_copy(out2_hbm.at[hx_vn.at[pl.ds(0, 6 * C)]],
                              rows_vn[s], sems[s]).wait()

    fire(0, 0)

    def chunk_pair(cc, carry):
        for bset in range(2):
            c = cc * 2 + bset
            fire(c + 1, 1 - bset)
            drain(bset)

            def group_body(g, gcarry):
                row0 = g * 16
                goff = pl.multiple_of(c * C + row0, 16)
                sl = pl.ds(goff, 16)
                pu = ((idx_u[sl] >> (RSH - 1)) & 1) * D
                pv = ((idx_v[sl] >> (RSH - 1)) & 1) * D
                pn = [((idx_n[pl.ds(k * BPW + goff, 16)] >> (RSH - 1)) & 1)
                      * D for k in range(K)]
                accp = jnp.zeros((16,), jnp.float32)
                accn = [jnp.zeros((16,), jnp.float32) for _ in range(K)]
                for rr in range(16):
                    r = row0 + rr
                    u = [rows_u[bset][r, pl.ds(pu[rr] + 16 * j, 16)]
                         for j in range(D // 16)]
                    v = [rows_vn[bset][r, pl.ds(pv[rr] + 16 * j, 16)]
                         for j in range(D // 16)]
                    s = u[0] * v[0] + u[1] * v[1] + u[2] * v[2] + u[3] * v[3]
                    accp = jnp.where(lane == rr, jnp.sum(s), accp)
                    for k in range(K):
                        n = [rows_vn[bset][(k + 1) * C + r,
                                           pl.ds(pn[k][rr] + 16 * j, 16)]
                             for j in range(D // 16)]
                        t = (u[0] * n[0] + u[1] * n[1] + u[2] * n[2]
                             + u[3] * n[3])
                        accn[k] = jnp.where(lane == rr, jnp.sum(t), accn[k])
                ps_buf[sl] = accp
                for k in range(K):
                    ns_buf[pl.ds(k * BPW + goff, 16)] = accn[k]
                return gcarry

            lax.fori_loop(0, C // 16, group_body, 0, unroll=False)
        return carry

    lax.fori_loop(0, NCH // 2, chunk_pair, 0, unroll=False)
    drain(0)

    pltpu.sync_copy(ps_buf, ps_hbm.at[pl.ds(base, BPW)])
    for k in range(K):
        pltpu.sync_copy(ns_buf.at[pl.ds(k * BPW, BPW)],
                        ns_hbm.at[pl.ds(k * B + base, BPW)])


@functools.partial(
    pl.kernel,
    out_type=[
        jax.ShapeDtypeStruct((B,), jnp.float32),
        jax.ShapeDtypeStruct((K * B,), jnp.float32),
    ],
    mesh=plsc.VectorSubcoreMesh(core_axis_name="c", subcore_axis_name="s",
                                num_cores=NC, num_subcores=NS),
    compiler_params=pltpu.CompilerParams(needs_layout_passes=False),
    scratch_types=[
        pltpu.VMEM((BPW,), jnp.int32),
        pltpu.VMEM((BPW,), jnp.int32),
        pltpu.VMEM((K * BPW,), jnp.int32),
        pltpu.VMEM((BPW,), jnp.int32),
        pltpu.VMEM((6 * BPW,), jnp.int32),
        [pltpu.VMEM((C, 2 * D), jnp.float32) for _ in range(2)],
        [pltpu.VMEM((6 * C, 2 * D), jnp.float32) for _ in range(2)],
        pltpu.VMEM((BPW,), jnp.float32),
        pltpu.VMEM((K * BPW,), jnp.float32),
        pltpu.SemaphoreType.DMA,
        pltpu.SemaphoreType.DMA,
        pltpu.SemaphoreType.DMA,
    ],
)
def _sc_scores(pos_u_hbm, pos_v_hbm, neg_t_hbm, in2_hbm, out2_hbm,
               ps_hbm, ns_hbm, *scratch):
    _sc_body(pos_u_hbm, pos_v_hbm, neg_t_hbm, in2_hbm, out2_hbm,
             ps_hbm, ns_hbm, *scratch)


RB = 32768
RH = RB // 2
RSH = RB.bit_length() - 1


def _relayout_body(xa_ref, xb_ref, o_ref):
    xa = xa_ref[...]
    xb = xb_ref[...]
    o_ref[...] = jnp.concatenate(
        [xa[:, 0:RH], xb[:, 0:RH], xa[:, RH:RB], xb[:, RH:RB]], axis=0).T


def _tc_relayout(xt):
    v = xt.shape[1]
    nb = pl.cdiv(v, RB)
    return pl.pallas_call(
        _relayout_body,
        grid=(nb,),
        in_specs=[pl.BlockSpec((D // 2, RB), lambda i: (0, i)),
                  pl.BlockSpec((D // 2, RB), lambda i: (1, i))],
        out_specs=pl.BlockSpec((RH, 2 * D), lambda i: (i, 0)),
        out_shape=jax.ShapeDtypeStruct((nb * RH, 2 * D), jnp.float32),
    )(xt, xt)


def _loss_body(ps_ref, ns_ref, out_ref):
    p = ps_ref[...]
    n = ns_ref[...]
    lsp = jax.nn.log_sigmoid(p)
    lsn = jax.nn.log_sigmoid(-n)
    total = jnp.sum(lsp) + jnp.sum(lsn)
    out_ref[0, 0] = -(total / B)


def _tc_loss(ps2d, ns2d):
    return pl.pallas_call(
        _loss_body,
        out_shape=jax.ShapeDtypeStruct((1, 1), jnp.float32),
        out_specs=pl.BlockSpec(memory_space=pltpu.SMEM),
    )(ps2d, ns2d)


def kernel(pos_u, pos_v, neg_v, in_embed, out_embed):
    neg_t = neg_v.astype(jnp.int32).T.reshape(K * B)
    in2 = _tc_relayout(in_embed.T)
    out2 = _tc_relayout(out_embed.T)
    ps, ns = _sc_scores(pos_u.astype(jnp.int32), pos_v.astype(jnp.int32),
                        neg_t, in2, out2)
    loss = _tc_loss(ps.reshape(B // 128, 128), ns.reshape(K * B // 128, 128))
    return loss[0, 0]
